# baseline (speedup 1.0000x reference)
import functools

import jax
import jax.numpy as jnp
from jax import lax
from jax.experimental import pallas as pl
from jax.experimental.pallas import tpu as pltpu
from jax.experimental.pallas import tpu_sc as plsc

NUM_CLIENTS = 100000
N_NODES = 10000
N_EDGES = 320000
D = 128

NC, NS = 2, 16
NW = NC * NS
N_PAD = 10240
B_W = N_PAD // NW
E_W = N_EDGES // NW
E_CHUNK = 128
E_PAD = 327680
E_WP = E_PAD // NW
E_STEPS = E_WP // E_CHUNK
N_ACC = 10240
ROWS_W = N_ACC // NS
DUMMY_DST = N_ACC - 8

_mesh = plsc.VectorSubcoreMesh(core_axis_name="c", subcore_axis_name="s",
                               num_cores=NC, num_subcores=NS)


def _embed_body(cidx_hbm, iidx_hbm, mask_hbm, ctab_hbm, itab_hbm, x_hbm,
                cidx_v, iidx_v, bufc, bufi, mrow, sem):
    w = lax.axis_index("s") * NC + lax.axis_index("c")
    base = w * B_W
    pltpu.sync_copy(cidx_hbm.at[pl.ds(base, B_W)], cidx_v)
    pltpu.sync_copy(iidx_hbm.at[pl.ds(base, B_W)], iidx_v)
    pltpu.async_copy(ctab_hbm.at[cidx_v], bufc, sem).wait()
    pltpu.async_copy(itab_hbm.at[iidx_v], bufi, sem).wait()
    pltpu.sync_copy(mask_hbm.at[pl.ds(base, B_W)], mrow)

    def row(r, _):
        for j in range(D // 16):
            sl = pl.ds(j * 16, 16)
            m = mrow[r, sl]
            bufc[r, sl] = bufi[r, sl] + m * (bufc[r, sl] - bufi[r, sl])
        return 0

    lax.fori_loop(0, B_W, row, 0)
    pltpu.sync_copy(bufc, x_hbm.at[pl.ds(base, B_W)])


_embed = pl.kernel(
    _embed_body,
    out_type=jax.ShapeDtypeStruct((N_PAD, D), jnp.float32),
    mesh=_mesh,
    scratch_types=[
        pltpu.VMEM((B_W,), jnp.int32),
        pltpu.VMEM((B_W,), jnp.int32),
        pltpu.VMEM((B_W, D), jnp.float32),
        pltpu.VMEM((B_W, D), jnp.float32),
        pltpu.VMEM((B_W, D), jnp.float32),
        pltpu.SemaphoreType.DMA,
    ],
)


def _deg_body(dstr_hbm, deg_hbm, didx, zeros_v, ones_v, deg_sh, sem):
    c = lax.axis_index("c")
    s = lax.axis_index("s")
    w = s * NC + c

    def init_row(r, _):
        for j in range(D // 16):
            zeros_v[r, pl.ds(j * 16, 16)] = jnp.zeros((16,), jnp.float32)
            ones_v[r, pl.ds(j * 16, 16)] = jnp.ones((16,), jnp.float32)
        return 0

    lax.fori_loop(0, E_CHUNK, init_row, 0)

    def zero_copy(k, _):
        off = s * ROWS_W + k * 128
        pltpu.sync_copy(zeros_v, deg_sh.at[pl.ds(off, 128)])
        return 0

    lax.fori_loop(0, ROWS_W // 128, zero_copy, 0)

    plsc.subcore_barrier()

    for h in range(2):
        pltpu.sync_copy(dstr_hbm.at[w, pl.ds(h * 40, 40)], didx)
        for g in range(5):
            for j in range(8):
                pltpu.async_copy(ones_v, deg_sh.at[didx.at[g * 8 + j]],
                                 sem, add=True)
            for j in range(8):
                pltpu.make_async_copy(ones_v, deg_sh.at[didx.at[g * 8]],
                                      sem).wait()

    plsc.subcore_barrier()

    off = s * ROWS_W
    pltpu.sync_copy(deg_sh.at[pl.ds(off, ROWS_W)],
                    deg_hbm.at[c, pl.ds(off, ROWS_W)])


_deg = pl.kernel(
    _deg_body,
    out_type=jax.ShapeDtypeStruct((NC, N_ACC, D), jnp.float32),
    mesh=_mesh,
    scratch_types=[
        pltpu.VMEM((40, E_CHUNK), jnp.int32),
        pltpu.VMEM((E_CHUNK, D), jnp.float32),
        pltpu.VMEM((E_CHUNK, D), jnp.float32),
        pltpu.VMEM_SHARED((N_ACC, D), jnp.float32),
        pltpu.SemaphoreType.DMA,
    ],
)


_HALF = E_STEPS // 2


def _agg_body(xn_hbm, srcr_hbm, dstr_hbm, acc_hbm,
              sidx, didx, rows, acc_sh, gsem, ssem0, ssem1):
    c = lax.axis_index("c")
    s = lax.axis_index("s")
    w = s * NC + c

    def zero_row(r, _):
        for j in range(D // 16):
            rows[0, r, pl.ds(j * 16, 16)] = jnp.zeros((16,), jnp.float32)
        return 0

    lax.fori_loop(0, E_CHUNK, zero_row, 0)

    def zero_copy(k, _):
        off = s * ROWS_W + k * 128
        pltpu.sync_copy(rows.at[0], acc_sh.at[pl.ds(off, 128)])
        return 0

    lax.fori_loop(0, ROWS_W // 128, zero_copy, 0)

    pltpu.sync_copy(srcr_hbm.at[w, pl.ds(0, _HALF)], sidx)
    pltpu.sync_copy(dstr_hbm.at[w, pl.ds(0, _HALF)], didx)

    plsc.subcore_barrier()

    pltpu.async_copy(xn_hbm.at[sidx.at[0]], rows.at[0], gsem)
    pending = [False, False]
    ssems = (ssem0, ssem1)
    for k in range(E_STEPS):
        b = k % 2
        j = k % _HALF
        pltpu.make_async_copy(xn_hbm.at[sidx.at[j]], rows.at[b], gsem).wait()
        pltpu.async_copy(rows.at[b], acc_sh.at[didx.at[j]], ssems[b],
                         add=True)
        pending[b] = True
        if k == _HALF - 1:
            for bb in (0, 1):
                if pending[bb]:
                    pltpu.make_async_copy(rows.at[bb], acc_sh.at[didx.at[0]],
                                          ssems[bb]).wait()
                    pending[bb] = False
            pltpu.sync_copy(srcr_hbm.at[w, pl.ds(_HALF, _HALF)], sidx)
            pltpu.sync_copy(dstr_hbm.at[w, pl.ds(_HALF, _HALF)], didx)
            pltpu.async_copy(xn_hbm.at[sidx.at[0]], rows.at[1 - b], gsem)
        elif k < E_STEPS - 1:
            if pending[1 - b]:
                pltpu.make_async_copy(rows.at[1 - b], acc_sh.at[didx.at[0]],
                                      ssems[1 - b]).wait()
                pending[1 - b] = False
            pltpu.async_copy(xn_hbm.at[sidx.at[(k + 1) % _HALF]],
                             rows.at[1 - b], gsem)
    for bb in (0, 1):
        if pending[bb]:
            pltpu.make_async_copy(rows.at[bb], acc_sh.at[didx.at[0]],
                                  ssems[bb]).wait()

    plsc.subcore_barrier()

    off = s * ROWS_W
    pltpu.sync_copy(acc_sh.at[pl.ds(off, ROWS_W)],
                    acc_hbm.at[c, pl.ds(off, ROWS_W)])


_agg = pl.kernel(
    _agg_body,
    out_type=jax.ShapeDtypeStruct((NC, N_ACC, D), jnp.float32),
    mesh=_mesh,
    scratch_types=[
        pltpu.VMEM((_HALF, E_CHUNK), jnp.int32),
        pltpu.VMEM((_HALF, E_CHUNK), jnp.int32),
        pltpu.VMEM((2, E_CHUNK, D), jnp.float32),
        pltpu.VMEM_SHARED((N_ACC, D), jnp.float32),
        pltpu.SemaphoreType.DMA,
        pltpu.SemaphoreType.DMA,
        pltpu.SemaphoreType.DMA,
    ],
)


_RB = 1000


def _mm1_body(x_ref, ws_ref, wn_ref, b_ref, xs_ref, xn_ref):
    x = x_ref[...]
    xs_ref[...] = jnp.dot(x, ws_ref[...],
                          preferred_element_type=jnp.float32) + b_ref[...]
    xn_ref[...] = jnp.dot(x, wn_ref[...], preferred_element_type=jnp.float32)


def _mm2_body(xs1_ref, acc_ref, deg_ref, ws_ref, wn_ref, b_ref,
              xs2_ref, xn2_ref):
    d = deg_ref[0] + deg_ref[1]
    rdeg = 1.0 / jnp.maximum(d[:, 0:1], 1.0)
    agg = (acc_ref[0] + acc_ref[1]) * rdeg
    h = jnp.maximum(xs1_ref[...] + agg, 0.0)
    xs2_ref[...] = jnp.dot(h, ws_ref[...],
                           preferred_element_type=jnp.float32) + b_ref[...]
    xn2_ref[...] = jnp.dot(h, wn_ref[...], preferred_element_type=jnp.float32)


def _fin_body(xs2_ref, acc_ref, deg_ref, out_ref):
    d = deg_ref[0] + deg_ref[1]
    rdeg = 1.0 / jnp.maximum(d[:, 0:1], 1.0)
    out_ref[...] = xs2_ref[...] + (acc_ref[0] + acc_ref[1]) * rdeg


def _row_spec(block):
    return pl.BlockSpec(block, lambda i: (0,) * len(block)) if block[0] != _RB \
        else pl.BlockSpec(block, lambda i: (i,) + (0,) * (len(block) - 1))


def _mm1(x, ws, wn, b):
    return pl.pallas_call(
        _mm1_body,
        grid=(N_NODES // _RB,),
        in_specs=[
            pl.BlockSpec((_RB, D), lambda i: (i, 0)),
            pl.BlockSpec((D, D), lambda i: (0, 0)),
            pl.BlockSpec((D, D), lambda i: (0, 0)),
            pl.BlockSpec((1, D), lambda i: (0, 0)),
        ],
        out_specs=[pl.BlockSpec((_RB, D), lambda i: (i, 0)),
                   pl.BlockSpec((_RB, D), lambda i: (i, 0))],
        out_shape=[jax.ShapeDtypeStruct((N_NODES, D), jnp.float32),
                   jax.ShapeDtypeStruct((N_NODES, D), jnp.float32)],
    )(x, ws, wn, b)


def _mm2(xs1, acc, deg, ws, wn, b):
    return pl.pallas_call(
        _mm2_body,
        grid=(N_NODES // _RB,),
        in_specs=[
            pl.BlockSpec((_RB, D), lambda i: (i, 0)),
            pl.BlockSpec((NC, _RB, D), lambda i: (0, i, 0)),
            pl.BlockSpec((NC, _RB, D), lambda i: (0, i, 0)),
            pl.BlockSpec((D, D), lambda i: (0, 0)),
            pl.BlockSpec((D, D), lambda i: (0, 0)),
            pl.BlockSpec((1, D), lambda i: (0, 0)),
        ],
        out_specs=[pl.BlockSpec((_RB, D), lambda i: (i, 0)),
                   pl.BlockSpec((_RB, D), lambda i: (i, 0))],
        out_shape=[jax.ShapeDtypeStruct((N_NODES, D), jnp.float32),
                   jax.ShapeDtypeStruct((N_NODES, D), jnp.float32)],
    )(xs1, acc, deg, ws, wn, b)


def _fin(xs2, acc, deg):
    return pl.pallas_call(
        _fin_body,
        grid=(N_NODES // _RB,),
        in_specs=[
            pl.BlockSpec((_RB, D), lambda i: (i, 0)),
            pl.BlockSpec((NC, _RB, D), lambda i: (0, i, 0)),
            pl.BlockSpec((NC, _RB, D), lambda i: (0, i, 0)),
        ],
        out_specs=pl.BlockSpec((_RB, D), lambda i: (i, 0)),
        out_shape=jax.ShapeDtypeStruct((N_NODES, D), jnp.float32),
    )(xs2, acc, deg)


def kernel(node_ids, edge_index, client_table, item_table,
           W_self1, W_neigh1, b1, W_self2, W_neigh2, b2):
    ids = jnp.concatenate(
        [node_ids, jnp.zeros((N_PAD - N_NODES,), jnp.int32)])
    is_c = ids < NUM_CLIENTS
    cidx = jnp.where(is_c, ids, 0)
    iidx = jnp.where(is_c, 0, ids - NUM_CLIENTS)
    maskf = jnp.broadcast_to(is_c[:, None], (N_PAD, D)).astype(jnp.float32)

    x_pad = _embed(cidx, iidx, maskf, client_table, item_table)
    x = x_pad[:N_NODES]

    pad_e = E_PAD - N_EDGES
    srcr = jnp.concatenate(
        [edge_index[0], jnp.zeros((pad_e,), jnp.int32)]
    ).reshape(NW, E_STEPS, E_CHUNK)
    dstr = jnp.concatenate(
        [edge_index[1], jnp.full((pad_e,), DUMMY_DST, jnp.int32)]
    ).reshape(NW, E_STEPS, E_CHUNK)

    b1r = b1.reshape(1, D)
    b2r = b2.reshape(1, D)

    _k, _v = jax.lax.sort([edge_index[1], edge_index[0]], num_keys=1)
    srcr = srcr + jnp.minimum(jnp.minimum(_k[0], _v[-1]), 0)

    deg = _deg(dstr)
    xs1, xn1 = _mm1(x, W_self1, W_neigh1, b1r)
    acc1 = _agg(xn1, srcr, dstr)
    xs2, xn2 = _mm2(xs1, acc1, deg, W_self2, W_neigh2, b2r)
    acc2 = _agg(xn2, srcr, dstr)
    return _fin(xs2, acc2, deg)


# speedup vs baseline: 1.0013x; 1.0013x over previous
import functools

import jax
import jax.numpy as jnp
from jax import lax
from jax.experimental import pallas as pl
from jax.experimental.pallas import tpu as pltpu
from jax.experimental.pallas import tpu_sc as plsc

NUM_CLIENTS = 100000
N_NODES = 10000
N_EDGES = 320000
D = 128

NC, NS = 2, 16
NW = NC * NS
N_PAD = 10240
B_W = N_PAD // NW
E_W = N_EDGES // NW
E_CHUNK = 128
E_PAD = 327680
E_WP = E_PAD // NW
E_STEPS = E_WP // E_CHUNK
N_ACC = 10240
ROWS_W = N_ACC // NS
DUMMY_DST = N_ACC - 8

_mesh = plsc.VectorSubcoreMesh(core_axis_name="c", subcore_axis_name="s",
                               num_cores=NC, num_subcores=NS)


def _embed_body(cidx_hbm, iidx_hbm, mask_hbm, ctab_hbm, itab_hbm, x_hbm,
                cidx_v, iidx_v, bufc, bufi, mrow, sem):
    w = lax.axis_index("s") * NC + lax.axis_index("c")
    base = w * B_W
    pltpu.sync_copy(cidx_hbm.at[pl.ds(base, B_W)], cidx_v)
    pltpu.sync_copy(iidx_hbm.at[pl.ds(base, B_W)], iidx_v)
    pltpu.async_copy(ctab_hbm.at[cidx_v], bufc, sem).wait()
    pltpu.async_copy(itab_hbm.at[iidx_v], bufi, sem).wait()
    pltpu.sync_copy(mask_hbm.at[pl.ds(base, B_W)], mrow)

    def row(r, _):
        for j in range(D // 16):
            sl = pl.ds(j * 16, 16)
            m = mrow[r, sl]
            bufc[r, sl] = bufi[r, sl] + m * (bufc[r, sl] - bufi[r, sl])
        return 0

    lax.fori_loop(0, B_W, row, 0)
    pltpu.sync_copy(bufc, x_hbm.at[pl.ds(base, B_W)])


_embed = pl.kernel(
    _embed_body,
    out_type=jax.ShapeDtypeStruct((N_PAD, D), jnp.float32),
    mesh=_mesh,
    scratch_types=[
        pltpu.VMEM((B_W,), jnp.int32),
        pltpu.VMEM((B_W,), jnp.int32),
        pltpu.VMEM((B_W, D), jnp.float32),
        pltpu.VMEM((B_W, D), jnp.float32),
        pltpu.VMEM((B_W, D), jnp.float32),
        pltpu.SemaphoreType.DMA,
    ],
)


def _deg_body(dstr_hbm, deg_hbm, didx, zeros_v, ones_v, deg_sh, sem):
    c = lax.axis_index("c")
    s = lax.axis_index("s")
    w = s * NC + c

    def init_row(r, _):
        for j in range(D // 16):
            zeros_v[r, pl.ds(j * 16, 16)] = jnp.zeros((16,), jnp.float32)
            ones_v[r, pl.ds(j * 16, 16)] = jnp.ones((16,), jnp.float32)
        return 0

    lax.fori_loop(0, E_CHUNK, init_row, 0)

    def zero_copy(k, _):
        off = s * ROWS_W + k * 128
        pltpu.sync_copy(zeros_v, deg_sh.at[pl.ds(off, 128)])
        return 0

    lax.fori_loop(0, ROWS_W // 128, zero_copy, 0)

    plsc.subcore_barrier()

    for h in range(2):
        pltpu.sync_copy(dstr_hbm.at[w, pl.ds(h * 40, 40)], didx)
        for g in range(5):
            for j in range(8):
                pltpu.async_copy(ones_v, deg_sh.at[didx.at[g * 8 + j]],
                                 sem, add=True)
            for j in range(8):
                pltpu.make_async_copy(ones_v, deg_sh.at[didx.at[g * 8]],
                                      sem).wait()

    plsc.subcore_barrier()

    off = s * ROWS_W
    pltpu.sync_copy(deg_sh.at[pl.ds(off, ROWS_W)],
                    deg_hbm.at[c, pl.ds(off, ROWS_W)])


_deg = pl.kernel(
    _deg_body,
    out_type=jax.ShapeDtypeStruct((NC, N_ACC, D), jnp.float32),
    mesh=_mesh,
    scratch_types=[
        pltpu.VMEM((40, E_CHUNK), jnp.int32),
        pltpu.VMEM((E_CHUNK, D), jnp.float32),
        pltpu.VMEM((E_CHUNK, D), jnp.float32),
        pltpu.VMEM_SHARED((N_ACC, D), jnp.float32),
        pltpu.SemaphoreType.DMA,
    ],
)


_HALF = E_STEPS // 2


def _agg_body(xn_hbm, srcr_hbm, dstr_hbm, acc_hbm,
              sidx, didx, rows, acc_sh, gsem, ssem0, ssem1):
    c = lax.axis_index("c")
    s = lax.axis_index("s")
    w = s * NC + c

    def zero_row(r, _):
        for j in range(D // 16):
            rows[0, r, pl.ds(j * 16, 16)] = jnp.zeros((16,), jnp.float32)
        return 0

    lax.fori_loop(0, E_CHUNK, zero_row, 0)

    def zero_copy(k, _):
        off = s * ROWS_W + k * 128
        pltpu.sync_copy(rows.at[0], acc_sh.at[pl.ds(off, 128)])
        return 0

    lax.fori_loop(0, ROWS_W // 128, zero_copy, 0)

    pltpu.sync_copy(srcr_hbm.at[w, pl.ds(0, _HALF)], sidx)
    pltpu.sync_copy(dstr_hbm.at[w, pl.ds(0, _HALF)], didx)

    plsc.subcore_barrier()

    pltpu.async_copy(xn_hbm.at[sidx.at[0]], rows.at[0], gsem)
    pending = [False, False]
    ssems = (ssem0, ssem1)
    for k in range(E_STEPS):
        b = k % 2
        j = k % _HALF
        pltpu.make_async_copy(xn_hbm.at[sidx.at[j]], rows.at[b], gsem).wait()
        pltpu.async_copy(rows.at[b], acc_sh.at[didx.at[j]], ssems[b],
                         add=True)
        pending[b] = True
        if k == _HALF - 1:
            for bb in (0, 1):
                if pending[bb]:
                    pltpu.make_async_copy(rows.at[bb], acc_sh.at[didx.at[0]],
                                          ssems[bb]).wait()
                    pending[bb] = False
            pltpu.sync_copy(srcr_hbm.at[w, pl.ds(_HALF, _HALF)], sidx)
            pltpu.sync_copy(dstr_hbm.at[w, pl.ds(_HALF, _HALF)], didx)
            pltpu.async_copy(xn_hbm.at[sidx.at[0]], rows.at[1 - b], gsem)
        elif k < E_STEPS - 1:
            if pending[1 - b]:
                pltpu.make_async_copy(rows.at[1 - b], acc_sh.at[didx.at[0]],
                                      ssems[1 - b]).wait()
                pending[1 - b] = False
            pltpu.async_copy(xn_hbm.at[sidx.at[(k + 1) % _HALF]],
                             rows.at[1 - b], gsem)
    for bb in (0, 1):
        if pending[bb]:
            pltpu.make_async_copy(rows.at[bb], acc_sh.at[didx.at[0]],
                                  ssems[bb]).wait()

    plsc.subcore_barrier()

    off = s * ROWS_W
    pltpu.sync_copy(acc_sh.at[pl.ds(off, ROWS_W)],
                    acc_hbm.at[c, pl.ds(off, ROWS_W)])


_agg = pl.kernel(
    _agg_body,
    out_type=jax.ShapeDtypeStruct((NC, N_ACC, D), jnp.float32),
    mesh=_mesh,
    scratch_types=[
        pltpu.VMEM((_HALF, E_CHUNK), jnp.int32),
        pltpu.VMEM((_HALF, E_CHUNK), jnp.int32),
        pltpu.VMEM((2, E_CHUNK, D), jnp.float32),
        pltpu.VMEM_SHARED((N_ACC, D), jnp.float32),
        pltpu.SemaphoreType.DMA,
        pltpu.SemaphoreType.DMA,
        pltpu.SemaphoreType.DMA,
    ],
)


_RB = 1000


def _mm1_body(x_ref, ws_ref, wn_ref, b_ref, xs_ref, xn_ref):
    x = x_ref[...]
    xs_ref[...] = jnp.dot(x, ws_ref[...],
                          preferred_element_type=jnp.float32) + b_ref[...]
    xn_ref[...] = jnp.dot(x, wn_ref[...], preferred_element_type=jnp.float32)


def _mm2_body(xs1_ref, acc_ref, deg_ref, ws_ref, wn_ref, b_ref,
              xs2_ref, xn2_ref):
    d = deg_ref[0] + deg_ref[1]
    rdeg = 1.0 / jnp.maximum(d[:, 0:1], 1.0)
    agg = (acc_ref[0] + acc_ref[1]) * rdeg
    h = jnp.maximum(xs1_ref[...] + agg, 0.0)
    xs2_ref[...] = jnp.dot(h, ws_ref[...],
                           preferred_element_type=jnp.float32) + b_ref[...]
    xn2_ref[...] = jnp.dot(h, wn_ref[...], preferred_element_type=jnp.float32)


def _fin_body(xs2_ref, acc_ref, deg_ref, out_ref):
    d = deg_ref[0] + deg_ref[1]
    rdeg = 1.0 / jnp.maximum(d[:, 0:1], 1.0)
    out_ref[...] = xs2_ref[...] + (acc_ref[0] + acc_ref[1]) * rdeg


def _row_spec(block):
    return pl.BlockSpec(block, lambda i: (0,) * len(block)) if block[0] != _RB \
        else pl.BlockSpec(block, lambda i: (i,) + (0,) * (len(block) - 1))


def _mm1(x, ws, wn, b):
    return pl.pallas_call(
        _mm1_body,
        grid=(N_NODES // _RB,),
        in_specs=[
            pl.BlockSpec((_RB, D), lambda i: (i, 0)),
            pl.BlockSpec((D, D), lambda i: (0, 0)),
            pl.BlockSpec((D, D), lambda i: (0, 0)),
            pl.BlockSpec((1, D), lambda i: (0, 0)),
        ],
        out_specs=[pl.BlockSpec((_RB, D), lambda i: (i, 0)),
                   pl.BlockSpec((_RB, D), lambda i: (i, 0))],
        out_shape=[jax.ShapeDtypeStruct((N_NODES, D), jnp.float32),
                   jax.ShapeDtypeStruct((N_NODES, D), jnp.float32)],
    )(x, ws, wn, b)


def _mm2(xs1, acc, deg, ws, wn, b):
    return pl.pallas_call(
        _mm2_body,
        grid=(N_NODES // _RB,),
        in_specs=[
            pl.BlockSpec((_RB, D), lambda i: (i, 0)),
            pl.BlockSpec((NC, _RB, D), lambda i: (0, i, 0)),
            pl.BlockSpec((NC, _RB, D), lambda i: (0, i, 0)),
            pl.BlockSpec((D, D), lambda i: (0, 0)),
            pl.BlockSpec((D, D), lambda i: (0, 0)),
            pl.BlockSpec((1, D), lambda i: (0, 0)),
        ],
        out_specs=[pl.BlockSpec((_RB, D), lambda i: (i, 0)),
                   pl.BlockSpec((_RB, D), lambda i: (i, 0))],
        out_shape=[jax.ShapeDtypeStruct((N_NODES, D), jnp.float32),
                   jax.ShapeDtypeStruct((N_NODES, D), jnp.float32)],
    )(xs1, acc, deg, ws, wn, b)


def _fin(xs2, acc, deg):
    return pl.pallas_call(
        _fin_body,
        grid=(N_NODES // _RB,),
        in_specs=[
            pl.BlockSpec((_RB, D), lambda i: (i, 0)),
            pl.BlockSpec((NC, _RB, D), lambda i: (0, i, 0)),
            pl.BlockSpec((NC, _RB, D), lambda i: (0, i, 0)),
        ],
        out_specs=pl.BlockSpec((_RB, D), lambda i: (i, 0)),
        out_shape=jax.ShapeDtypeStruct((N_NODES, D), jnp.float32),
    )(xs2, acc, deg)


def kernel(node_ids, edge_index, client_table, item_table,
           W_self1, W_neigh1, b1, W_self2, W_neigh2, b2):
    ids = jnp.concatenate(
        [node_ids, jnp.zeros((N_PAD - N_NODES,), jnp.int32)])
    is_c = ids < NUM_CLIENTS
    cidx = jnp.where(is_c, ids, 0)
    iidx = jnp.where(is_c, 0, ids - NUM_CLIENTS)
    maskf = jnp.broadcast_to(is_c[:, None], (N_PAD, D)).astype(jnp.float32)

    x_pad = _embed(cidx, iidx, maskf, client_table, item_table)
    x = x_pad[:N_NODES]

    pad_e = E_PAD - N_EDGES
    dst_s, src_s = jax.lax.sort([edge_index[1], edge_index[0]], num_keys=1)
    srcr = jnp.concatenate(
        [src_s, jnp.zeros((pad_e,), jnp.int32)]
    ).reshape(NW, E_STEPS, E_CHUNK)
    dstr = jnp.concatenate(
        [dst_s, jnp.full((pad_e,), DUMMY_DST, jnp.int32)]
    ).reshape(NW, E_STEPS, E_CHUNK)

    b1r = b1.reshape(1, D)
    b2r = b2.reshape(1, D)

    deg = _deg(dstr)
    xs1, xn1 = _mm1(x, W_self1, W_neigh1, b1r)
    acc1 = _agg(xn1, srcr, dstr)
    xs2, xn2 = _mm2(xs1, acc1, deg, W_self2, W_neigh2, b2r)
    acc2 = _agg(xn2, srcr, dstr)
    return _fin(xs2, acc2, deg)


# speedup vs baseline: 1.1574x; 1.1559x over previous
import functools

import jax
import jax.numpy as jnp
from jax import lax
from jax.experimental import pallas as pl
from jax.experimental.pallas import tpu as pltpu
from jax.experimental.pallas import tpu_sc as plsc

NUM_CLIENTS = 100000
N_NODES = 10000
N_EDGES = 320000
D = 128

NC, NS = 2, 16
NW = NC * NS
N_PAD = 10240
B_W = N_PAD // NW
E_W = N_EDGES // NW
E_CHUNK = 128
E_PAD = 327680
E_WP = E_PAD // NW
E_STEPS = E_WP // E_CHUNK
N_ACC = 10240
ROWS_W = N_ACC // NS
DUMMY_DST = N_ACC - 8

_mesh = plsc.VectorSubcoreMesh(core_axis_name="c", subcore_axis_name="s",
                               num_cores=NC, num_subcores=NS)


_EC2 = 32


def _prep_body(cidx_hbm, iidx_hbm, mask_hbm, ctab_hbm, itab_hbm, dstr_hbm,
               x_hbm, deg_hbm,
               cidx_v, iidx_v, bufc, bufi, mrow, didx, ones_v, deg_sh,
               gsem, dsem):
    c = lax.axis_index("c")
    s = lax.axis_index("s")
    w = s * NC + c
    base = w * B_W

    def zrow(r, _):
        for j in range(D // 16):
            ones_v[r, pl.ds(j * 16, 16)] = jnp.zeros((16,), jnp.float32)
        return 0

    lax.fori_loop(0, E_CHUNK, zrow, 0)
    for k in range(ROWS_W // 128):
        pltpu.async_copy(ones_v, deg_sh.at[pl.ds(s * ROWS_W + k * 128, 128)],
                         dsem)
    pltpu.sync_copy(cidx_hbm.at[pl.ds(base, B_W)], cidx_v)
    pltpu.sync_copy(iidx_hbm.at[pl.ds(base, B_W)], iidx_v)
    for k in range(ROWS_W // 128):
        pltpu.make_async_copy(ones_v, deg_sh.at[pl.ds(s * ROWS_W, 128)],
                              dsem).wait()

    def orow(r, _):
        for j in range(D // 16):
            ones_v[r, pl.ds(j * 16, 16)] = jnp.ones((16,), jnp.float32)
        return 0

    lax.fori_loop(0, E_CHUNK, orow, 0)
    pltpu.sync_copy(dstr_hbm.at[w, pl.ds(0, _HALF)], didx)
    plsc.subcore_barrier()

    for h in range(2):
        if h == 1:
            pltpu.sync_copy(dstr_hbm.at[w, pl.ds(_HALF, _HALF)], didx)
        for g in range(5):
            for j in range(8):
                pltpu.async_copy(ones_v, deg_sh.at[didx.at[g * 8 + j]],
                                 dsem, add=True)
            eb = (h * 5 + g) * _EC2
            pltpu.async_copy(ctab_hbm.at[cidx_v.at[pl.ds(eb, _EC2)]],
                             bufc, gsem).wait()
            pltpu.async_copy(itab_hbm.at[iidx_v.at[pl.ds(eb, _EC2)]],
                             bufi, gsem).wait()
            pltpu.sync_copy(mask_hbm.at[pl.ds(base + eb, _EC2)], mrow)

            def row(r, _):
                for j2 in range(D // 16):
                    sl = pl.ds(j2 * 16, 16)
                    m = mrow[r, sl]
                    bufc[r, sl] = bufi[r, sl] + m * (bufc[r, sl] - bufi[r, sl])
                return 0

            lax.fori_loop(0, _EC2, row, 0)
            pltpu.sync_copy(bufc, x_hbm.at[pl.ds(base + eb, _EC2)])
            for j in range(8):
                pltpu.make_async_copy(ones_v, deg_sh.at[didx.at[g * 8]],
                                      dsem).wait()

    plsc.subcore_barrier()

    off = s * ROWS_W
    pltpu.sync_copy(deg_sh.at[pl.ds(off, ROWS_W)],
                    deg_hbm.at[c, pl.ds(off, ROWS_W)])


_prep = pl.kernel(
    _prep_body,
    out_type=(jax.ShapeDtypeStruct((N_PAD, D), jnp.float32),
              jax.ShapeDtypeStruct((NC, N_ACC, D), jnp.float32)),
    mesh=_mesh,
    scratch_types=[
        pltpu.VMEM((B_W,), jnp.int32),
        pltpu.VMEM((B_W,), jnp.int32),
        pltpu.VMEM((_EC2, D), jnp.float32),
        pltpu.VMEM((_EC2, D), jnp.float32),
        pltpu.VMEM((_EC2, D), jnp.float32),
        pltpu.VMEM((40, E_CHUNK), jnp.int32),
        pltpu.VMEM((E_CHUNK, D), jnp.float32),
        pltpu.VMEM_SHARED((N_ACC, D), jnp.float32),
        pltpu.SemaphoreType.DMA,
        pltpu.SemaphoreType.DMA,
    ],
)


_HALF = E_STEPS // 2


def _agg_body(xn_hbm, srcr_hbm, dstr_hbm, acc_hbm,
              sidx, didx, rows, acc_sh, gsem, ssem0, ssem1):
    c = lax.axis_index("c")
    s = lax.axis_index("s")
    w = s * NC + c

    def zero_row(r, _):
        for j in range(D // 16):
            rows[0, r, pl.ds(j * 16, 16)] = jnp.zeros((16,), jnp.float32)
        return 0

    lax.fori_loop(0, E_CHUNK, zero_row, 0)

    def zero_copy(k, _):
        off = s * ROWS_W + k * 128
        pltpu.sync_copy(rows.at[0], acc_sh.at[pl.ds(off, 128)])
        return 0

    lax.fori_loop(0, ROWS_W // 128, zero_copy, 0)

    pltpu.sync_copy(srcr_hbm.at[w, pl.ds(0, _HALF)], sidx)
    pltpu.sync_copy(dstr_hbm.at[w, pl.ds(0, _HALF)], didx)

    plsc.subcore_barrier()

    pltpu.async_copy(xn_hbm.at[sidx.at[0]], rows.at[0], gsem)
    pending = [False, False]
    ssems = (ssem0, ssem1)
    for k in range(E_STEPS):
        b = k % 2
        j = k % _HALF
        pltpu.make_async_copy(xn_hbm.at[sidx.at[j]], rows.at[b], gsem).wait()
        pltpu.async_copy(rows.at[b], acc_sh.at[didx.at[j]], ssems[b],
                         add=True)
        pending[b] = True
        if k == _HALF - 1:
            for bb in (0, 1):
                if pending[bb]:
                    pltpu.make_async_copy(rows.at[bb], acc_sh.at[didx.at[0]],
                                          ssems[bb]).wait()
                    pending[bb] = False
            pltpu.sync_copy(srcr_hbm.at[w, pl.ds(_HALF, _HALF)], sidx)
            pltpu.sync_copy(dstr_hbm.at[w, pl.ds(_HALF, _HALF)], didx)
            pltpu.async_copy(xn_hbm.at[sidx.at[0]], rows.at[1 - b], gsem)
        elif k < E_STEPS - 1:
            if pending[1 - b]:
                pltpu.make_async_copy(rows.at[1 - b], acc_sh.at[didx.at[0]],
                                      ssems[1 - b]).wait()
                pending[1 - b] = False
            pltpu.async_copy(xn_hbm.at[sidx.at[(k + 1) % _HALF]],
                             rows.at[1 - b], gsem)
    for bb in (0, 1):
        if pending[bb]:
            pltpu.make_async_copy(rows.at[bb], acc_sh.at[didx.at[0]],
                                  ssems[bb]).wait()

    plsc.subcore_barrier()

    off = s * ROWS_W
    pltpu.sync_copy(acc_sh.at[pl.ds(off, ROWS_W)],
                    acc_hbm.at[c, pl.ds(off, ROWS_W)])


_agg = pl.kernel(
    _agg_body,
    out_type=jax.ShapeDtypeStruct((NC, N_ACC, D), jnp.float32),
    mesh=_mesh,
    scratch_types=[
        pltpu.VMEM((_HALF, E_CHUNK), jnp.int32),
        pltpu.VMEM((_HALF, E_CHUNK), jnp.int32),
        pltpu.VMEM((2, E_CHUNK, D), jnp.float32),
        pltpu.VMEM_SHARED((N_ACC, D), jnp.float32),
        pltpu.SemaphoreType.DMA,
        pltpu.SemaphoreType.DMA,
        pltpu.SemaphoreType.DMA,
    ],
)


_RB = 1000


def _mm1_body(x_ref, ws_ref, wn_ref, b_ref, xs_ref, xn_ref):
    x = x_ref[...]
    xs_ref[...] = jnp.dot(x, ws_ref[...],
                          preferred_element_type=jnp.float32) + b_ref[...]
    xn_ref[...] = jnp.dot(x, wn_ref[...], preferred_element_type=jnp.float32)


def _mm2_body(xs1_ref, acc_ref, deg_ref, ws_ref, wn_ref, b_ref,
              xs2_ref, xn2_ref):
    d = deg_ref[0] + deg_ref[1]
    rdeg = 1.0 / jnp.maximum(d[:, 0:1], 1.0)
    agg = (acc_ref[0] + acc_ref[1]) * rdeg
    h = jnp.maximum(xs1_ref[...] + agg, 0.0)
    xs2_ref[...] = jnp.dot(h, ws_ref[...],
                           preferred_element_type=jnp.float32) + b_ref[...]
    xn2_ref[...] = jnp.dot(h, wn_ref[...], preferred_element_type=jnp.float32)


def _fin_body(xs2_ref, acc_ref, deg_ref, out_ref):
    d = deg_ref[0] + deg_ref[1]
    rdeg = 1.0 / jnp.maximum(d[:, 0:1], 1.0)
    out_ref[...] = xs2_ref[...] + (acc_ref[0] + acc_ref[1]) * rdeg


def _row_spec(block):
    return pl.BlockSpec(block, lambda i: (0,) * len(block)) if block[0] != _RB \
        else pl.BlockSpec(block, lambda i: (i,) + (0,) * (len(block) - 1))


def _mm1(x, ws, wn, b):
    return pl.pallas_call(
        _mm1_body,
        grid=(N_NODES // _RB,),
        in_specs=[
            pl.BlockSpec((_RB, D), lambda i: (i, 0)),
            pl.BlockSpec((D, D), lambda i: (0, 0)),
            pl.BlockSpec((D, D), lambda i: (0, 0)),
            pl.BlockSpec((1, D), lambda i: (0, 0)),
        ],
        out_specs=[pl.BlockSpec((_RB, D), lambda i: (i, 0)),
                   pl.BlockSpec((_RB, D), lambda i: (i, 0))],
        out_shape=[jax.ShapeDtypeStruct((N_NODES, D), jnp.float32),
                   jax.ShapeDtypeStruct((N_NODES, D), jnp.float32)],
    )(x, ws, wn, b)


def _mm2(xs1, acc, deg, ws, wn, b):
    return pl.pallas_call(
        _mm2_body,
        grid=(N_NODES // _RB,),
        in_specs=[
            pl.BlockSpec((_RB, D), lambda i: (i, 0)),
            pl.BlockSpec((NC, _RB, D), lambda i: (0, i, 0)),
            pl.BlockSpec((NC, _RB, D), lambda i: (0, i, 0)),
            pl.BlockSpec((D, D), lambda i: (0, 0)),
            pl.BlockSpec((D, D), lambda i: (0, 0)),
            pl.BlockSpec((1, D), lambda i: (0, 0)),
        ],
        out_specs=[pl.BlockSpec((_RB, D), lambda i: (i, 0)),
                   pl.BlockSpec((_RB, D), lambda i: (i, 0))],
        out_shape=[jax.ShapeDtypeStruct((N_NODES, D), jnp.float32),
                   jax.ShapeDtypeStruct((N_NODES, D), jnp.float32)],
    )(xs1, acc, deg, ws, wn, b)


def _fin(xs2, acc, deg):
    return pl.pallas_call(
        _fin_body,
        grid=(N_NODES // _RB,),
        in_specs=[
            pl.BlockSpec((_RB, D), lambda i: (i, 0)),
            pl.BlockSpec((NC, _RB, D), lambda i: (0, i, 0)),
            pl.BlockSpec((NC, _RB, D), lambda i: (0, i, 0)),
        ],
        out_specs=pl.BlockSpec((_RB, D), lambda i: (i, 0)),
        out_shape=jax.ShapeDtypeStruct((N_NODES, D), jnp.float32),
    )(xs2, acc, deg)


def kernel(node_ids, edge_index, client_table, item_table,
           W_self1, W_neigh1, b1, W_self2, W_neigh2, b2):
    ids = jnp.concatenate(
        [node_ids, jnp.zeros((N_PAD - N_NODES,), jnp.int32)])
    is_c = ids < NUM_CLIENTS
    cidx = jnp.where(is_c, ids, 0)
    iidx = jnp.where(is_c, 0, ids - NUM_CLIENTS)
    maskf = jnp.broadcast_to(is_c[:, None], (N_PAD, D)).astype(jnp.float32)

    pad_e = E_PAD - N_EDGES
    srcr = jnp.concatenate(
        [edge_index[0], jnp.zeros((pad_e,), jnp.int32)]
    ).reshape(NW, E_STEPS, E_CHUNK)
    dstr = jnp.concatenate(
        [edge_index[1], jnp.full((pad_e,), DUMMY_DST, jnp.int32)]
    ).reshape(NW, E_STEPS, E_CHUNK)

    b1r = b1.reshape(1, D)
    b2r = b2.reshape(1, D)

    x_pad, deg = _prep(cidx, iidx, maskf, client_table, item_table, dstr)
    x = x_pad[:N_NODES]
    xs1, xn1 = _mm1(x, W_self1, W_neigh1, b1r)
    acc1 = _agg(xn1, srcr, dstr)
    xs2, xn2 = _mm2(xs1, acc1, deg, W_self2, W_neigh2, b2r)
    acc2 = _agg(xn2, srcr, dstr)
    return _fin(xs2, acc2, deg)


# speedup vs baseline: 1.2308x; 1.0634x over previous
import functools

import jax
import jax.numpy as jnp
from jax import lax
from jax.experimental import pallas as pl
from jax.experimental.pallas import tpu as pltpu
from jax.experimental.pallas import tpu_sc as plsc

NUM_CLIENTS = 100000
N_NODES = 10000
N_EDGES = 320000
D = 128

NC, NS = 2, 16
NW = NC * NS
N_PAD = 10240
B_W = N_PAD // NW
E_W = N_EDGES // NW
E_CHUNK = 128
E_PAD = 327680
E_WP = E_PAD // NW
E_STEPS = E_WP // E_CHUNK
N_ACC = 10240
ROWS_W = N_ACC // NS
DUMMY_DST = N_ACC - 8

_mesh = plsc.VectorSubcoreMesh(core_axis_name="c", subcore_axis_name="s",
                               num_cores=NC, num_subcores=NS)


_EC2 = 32


def _prep_body(cidx_hbm, iidx_hbm, mask_hbm, ctab_hbm, itab_hbm, dstr_hbm,
               x_hbm, deg_hbm,
               cidx_v, iidx_v, bufc, bufi, mrow, didx, ones_v, deg_sh,
               gsem, dsem):
    c = lax.axis_index("c")
    s = lax.axis_index("s")
    w = s * NC + c
    base = w * B_W

    def zrow(r, _):
        for j in range(D // 16):
            ones_v[r, pl.ds(j * 16, 16)] = jnp.zeros((16,), jnp.float32)
        return 0

    lax.fori_loop(0, E_CHUNK, zrow, 0)
    for k in range(ROWS_W // 128):
        pltpu.async_copy(ones_v, deg_sh.at[pl.ds(s * ROWS_W + k * 128, 128)],
                         dsem)
    pltpu.sync_copy(cidx_hbm.at[pl.ds(base, B_W)], cidx_v)
    pltpu.sync_copy(iidx_hbm.at[pl.ds(base, B_W)], iidx_v)
    for k in range(ROWS_W // 128):
        pltpu.make_async_copy(ones_v, deg_sh.at[pl.ds(s * ROWS_W, 128)],
                              dsem).wait()

    def orow(r, _):
        for j in range(D // 16):
            ones_v[r, pl.ds(j * 16, 16)] = jnp.ones((16,), jnp.float32)
        return 0

    lax.fori_loop(0, E_CHUNK, orow, 0)
    pltpu.sync_copy(dstr_hbm.at[pl.ds(w * E_STEPS, E_STEPS // 2)], didx)
    plsc.subcore_barrier()

    for h in range(2):
        if h == 1:
            pltpu.sync_copy(
                dstr_hbm.at[pl.ds(w * E_STEPS + E_STEPS // 2, E_STEPS // 2)],
                didx)
        for g in range(5):
            for j in range(8):
                pltpu.async_copy(ones_v, deg_sh.at[didx.at[g * 8 + j]],
                                 dsem, add=True)
            eb = (h * 5 + g) * _EC2
            pltpu.async_copy(ctab_hbm.at[cidx_v.at[pl.ds(eb, _EC2)]],
                             bufc, gsem).wait()
            pltpu.async_copy(itab_hbm.at[iidx_v.at[pl.ds(eb, _EC2)]],
                             bufi, gsem).wait()
            pltpu.sync_copy(mask_hbm.at[pl.ds(base + eb, _EC2)], mrow)

            def row(r, _):
                for j2 in range(D // 16):
                    sl = pl.ds(j2 * 16, 16)
                    m = mrow[r, sl]
                    bufc[r, sl] = bufi[r, sl] + m * (bufc[r, sl] - bufi[r, sl])
                return 0

            lax.fori_loop(0, _EC2, row, 0)
            pltpu.sync_copy(bufc, x_hbm.at[pl.ds(base + eb, _EC2)])
            for j in range(8):
                pltpu.make_async_copy(ones_v, deg_sh.at[didx.at[g * 8]],
                                      dsem).wait()

    plsc.subcore_barrier()

    off = s * ROWS_W
    pltpu.sync_copy(deg_sh.at[pl.ds(off, ROWS_W)],
                    deg_hbm.at[c, pl.ds(off, ROWS_W)])


_prep = pl.kernel(
    _prep_body,
    out_type=(jax.ShapeDtypeStruct((N_PAD, D), jnp.float32),
              jax.ShapeDtypeStruct((NC, N_ACC, D), jnp.float32)),
    mesh=_mesh,
    scratch_types=[
        pltpu.VMEM((B_W,), jnp.int32),
        pltpu.VMEM((B_W,), jnp.int32),
        pltpu.VMEM((_EC2, D), jnp.float32),
        pltpu.VMEM((_EC2, D), jnp.float32),
        pltpu.VMEM((_EC2, D), jnp.float32),
        pltpu.VMEM((40, E_CHUNK), jnp.int32),
        pltpu.VMEM((E_CHUNK, D), jnp.float32),
        pltpu.VMEM_SHARED((N_ACC, D), jnp.float32),
        pltpu.SemaphoreType.DMA,
        pltpu.SemaphoreType.DMA,
    ],
)


_HALF = E_STEPS // 2

_N0 = 48
_N1 = 112


def _edge_pipeline(xn_hbm, srcr_hbm, dstr_hbm, sidx, didx, rows, acc_sh,
                   gsem, ssem0, ssem1, base, n):
    ssems = (ssem0, ssem1)
    pending = [False, False]
    for off in range(0, n, _HALF):
        g = min(_HALF, n - off)
        for bb in (0, 1):
            if pending[bb]:
                pltpu.make_async_copy(rows.at[bb], acc_sh.at[didx.at[0]],
                                      ssems[bb]).wait()
                pending[bb] = False
        pltpu.sync_copy(srcr_hbm.at[pl.ds(base + off, g)],
                        sidx.at[pl.ds(0, g)])
        pltpu.sync_copy(dstr_hbm.at[pl.ds(base + off, g)],
                        didx.at[pl.ds(0, g)])
        pltpu.async_copy(xn_hbm.at[sidx.at[0]], rows.at[off % 2], gsem)
        for j in range(g):
            k = off + j
            b = k % 2
            pltpu.make_async_copy(xn_hbm.at[sidx.at[j]], rows.at[b],
                                  gsem).wait()
            pltpu.async_copy(rows.at[b], acc_sh.at[didx.at[j]], ssems[b],
                             add=True)
            pending[b] = True
            if j < g - 1:
                if pending[1 - b]:
                    pltpu.make_async_copy(rows.at[1 - b],
                                          acc_sh.at[didx.at[0]],
                                          ssems[1 - b]).wait()
                    pending[1 - b] = False
                pltpu.async_copy(xn_hbm.at[sidx.at[j + 1]], rows.at[1 - b],
                                 gsem)
    for bb in (0, 1):
        if pending[bb]:
            pltpu.make_async_copy(rows.at[bb], acc_sh.at[didx.at[0]],
                                  ssems[bb]).wait()


def _agg_body(xn_hbm, srcr_hbm, dstr_hbm, acc_hbm,
              sidx, didx, rows, acc_sh, gsem, ssem0, ssem1):
    c = lax.axis_index("c")
    s = lax.axis_index("s")

    def zero_row(r, _):
        for j in range(D // 16):
            rows[0, r, pl.ds(j * 16, 16)] = jnp.zeros((16,), jnp.float32)
        return 0

    lax.fori_loop(0, E_CHUNK, zero_row, 0)

    def zero_copy(k, _):
        off = s * ROWS_W + k * 128
        pltpu.sync_copy(rows.at[0], acc_sh.at[pl.ds(off, 128)])
        return 0

    lax.fori_loop(0, ROWS_W // 128, zero_copy, 0)

    plsc.subcore_barrier()

    @pl.when(c == 0)
    def _():
        _edge_pipeline(xn_hbm, srcr_hbm, dstr_hbm, sidx, didx, rows, acc_sh,
                       gsem, ssem0, ssem1, s * _N0, _N0)

    @pl.when(c == 1)
    def _():
        _edge_pipeline(xn_hbm, srcr_hbm, dstr_hbm, sidx, didx, rows, acc_sh,
                       gsem, ssem0, ssem1, NS * _N0 + s * _N1, _N1)

    plsc.subcore_barrier()

    off = s * ROWS_W
    pltpu.sync_copy(acc_sh.at[pl.ds(off, ROWS_W)],
                    acc_hbm.at[c, pl.ds(off, ROWS_W)])


_agg = pl.kernel(
    _agg_body,
    out_type=jax.ShapeDtypeStruct((NC, N_ACC, D), jnp.float32),
    mesh=_mesh,
    scratch_types=[
        pltpu.VMEM((_HALF, E_CHUNK), jnp.int32),
        pltpu.VMEM((_HALF, E_CHUNK), jnp.int32),
        pltpu.VMEM((2, E_CHUNK, D), jnp.float32),
        pltpu.VMEM_SHARED((N_ACC, D), jnp.float32),
        pltpu.SemaphoreType.DMA,
        pltpu.SemaphoreType.DMA,
        pltpu.SemaphoreType.DMA,
    ],
)


_RB = 1000


def _mm1_body(x_ref, ws_ref, wn_ref, b_ref, xs_ref, xn_ref):
    x = x_ref[...]
    xs_ref[...] = jnp.dot(x, ws_ref[...],
                          preferred_element_type=jnp.float32) + b_ref[...]
    xn_ref[...] = jnp.dot(x, wn_ref[...], preferred_element_type=jnp.float32)


def _mm2_body(xs1_ref, acc_ref, deg_ref, ws_ref, wn_ref, b_ref,
              xs2_ref, xn2_ref):
    d = deg_ref[0] + deg_ref[1]
    rdeg = 1.0 / jnp.maximum(d[:, 0:1], 1.0)
    agg = (acc_ref[0] + acc_ref[1]) * rdeg
    h = jnp.maximum(xs1_ref[...] + agg, 0.0)
    xs2_ref[...] = jnp.dot(h, ws_ref[...],
                           preferred_element_type=jnp.float32) + b_ref[...]
    xn2_ref[...] = jnp.dot(h, wn_ref[...], preferred_element_type=jnp.float32)


def _fin_body(xs2_ref, acc_ref, deg_ref, out_ref):
    d = deg_ref[0] + deg_ref[1]
    rdeg = 1.0 / jnp.maximum(d[:, 0:1], 1.0)
    out_ref[...] = xs2_ref[...] + (acc_ref[0] + acc_ref[1]) * rdeg


def _row_spec(block):
    return pl.BlockSpec(block, lambda i: (0,) * len(block)) if block[0] != _RB \
        else pl.BlockSpec(block, lambda i: (i,) + (0,) * (len(block) - 1))


def _mm1(x, ws, wn, b):
    return pl.pallas_call(
        _mm1_body,
        grid=(N_NODES // _RB,),
        in_specs=[
            pl.BlockSpec((_RB, D), lambda i: (i, 0)),
            pl.BlockSpec((D, D), lambda i: (0, 0)),
            pl.BlockSpec((D, D), lambda i: (0, 0)),
            pl.BlockSpec((1, D), lambda i: (0, 0)),
        ],
        out_specs=[pl.BlockSpec((_RB, D), lambda i: (i, 0)),
                   pl.BlockSpec((_RB, D), lambda i: (i, 0))],
        out_shape=[jax.ShapeDtypeStruct((N_NODES, D), jnp.float32),
                   jax.ShapeDtypeStruct((N_NODES, D), jnp.float32)],
    )(x, ws, wn, b)


def _mm2(xs1, acc, deg, ws, wn, b):
    return pl.pallas_call(
        _mm2_body,
        grid=(N_NODES // _RB,),
        in_specs=[
            pl.BlockSpec((_RB, D), lambda i: (i, 0)),
            pl.BlockSpec((NC, _RB, D), lambda i: (0, i, 0)),
            pl.BlockSpec((NC, _RB, D), lambda i: (0, i, 0)),
            pl.BlockSpec((D, D), lambda i: (0, 0)),
            pl.BlockSpec((D, D), lambda i: (0, 0)),
            pl.BlockSpec((1, D), lambda i: (0, 0)),
        ],
        out_specs=[pl.BlockSpec((_RB, D), lambda i: (i, 0)),
                   pl.BlockSpec((_RB, D), lambda i: (i, 0))],
        out_shape=[jax.ShapeDtypeStruct((N_NODES, D), jnp.float32),
                   jax.ShapeDtypeStruct((N_NODES, D), jnp.float32)],
    )(xs1, acc, deg, ws, wn, b)


def _fin(xs2, acc, deg):
    return pl.pallas_call(
        _fin_body,
        grid=(N_NODES // _RB,),
        in_specs=[
            pl.BlockSpec((_RB, D), lambda i: (i, 0)),
            pl.BlockSpec((NC, _RB, D), lambda i: (0, i, 0)),
            pl.BlockSpec((NC, _RB, D), lambda i: (0, i, 0)),
        ],
        out_specs=pl.BlockSpec((_RB, D), lambda i: (i, 0)),
        out_shape=jax.ShapeDtypeStruct((N_NODES, D), jnp.float32),
    )(xs2, acc, deg)


def kernel(node_ids, edge_index, client_table, item_table,
           W_self1, W_neigh1, b1, W_self2, W_neigh2, b2):
    ids = jnp.concatenate(
        [node_ids, jnp.zeros((N_PAD - N_NODES,), jnp.int32)])
    is_c = ids < NUM_CLIENTS
    cidx = jnp.where(is_c, ids, 0)
    iidx = jnp.where(is_c, 0, ids - NUM_CLIENTS)
    maskf = jnp.broadcast_to(is_c[:, None], (N_PAD, D)).astype(jnp.float32)

    pad_e = E_PAD - N_EDGES
    srcr = jnp.concatenate(
        [edge_index[0], jnp.zeros((pad_e,), jnp.int32)]
    ).reshape(NW * E_STEPS, E_CHUNK)
    dstr = jnp.concatenate(
        [edge_index[1], jnp.full((pad_e,), DUMMY_DST, jnp.int32)]
    ).reshape(NW * E_STEPS, E_CHUNK)

    b1r = b1.reshape(1, D)
    b2r = b2.reshape(1, D)

    x_pad, deg = _prep(cidx, iidx, maskf, client_table, item_table, dstr)
    x = x_pad[:N_NODES]
    xs1, xn1 = _mm1(x, W_self1, W_neigh1, b1r)
    acc1 = _agg(xn1, srcr, dstr)
    xs2, xn2 = _mm2(xs1, acc1, deg, W_self2, W_neigh2, b2r)
    acc2 = _agg(xn2, srcr, dstr)
    return _fin(xs2, acc2, deg)


# speedup vs baseline: 1.3045x; 1.0599x over previous
import functools

import jax
import jax.numpy as jnp
from jax import lax
from jax.experimental import pallas as pl
from jax.experimental.pallas import tpu as pltpu
from jax.experimental.pallas import tpu_sc as plsc

NUM_CLIENTS = 100000
N_NODES = 10000
N_EDGES = 320000
D = 128

NC, NS = 2, 16
NW = NC * NS
N_PAD = 10240
B_W = N_PAD // NW
E_W = N_EDGES // NW
E_CHUNK = 128
E_PAD = 327680
E_WP = E_PAD // NW
E_STEPS = E_WP // E_CHUNK
N_ACC = 10240
ROWS_W = N_ACC // NS
DUMMY_DST = N_ACC - 8

_mesh = plsc.VectorSubcoreMesh(core_axis_name="c", subcore_axis_name="s",
                               num_cores=NC, num_subcores=NS)


_EC2 = 32


def _prep_body(cidx_hbm, iidx_hbm, mask_hbm, ctab_hbm, itab_hbm, dstr_hbm,
               x_hbm, deg_hbm,
               cidx_v, iidx_v, bufc, bufi, mrow, didx, ones_v, deg_sh,
               gsem, dsem):
    c = lax.axis_index("c")
    s = lax.axis_index("s")
    w = s * NC + c
    base = w * B_W

    def zrow(r, _):
        for j in range(D // 16):
            ones_v[r, pl.ds(j * 16, 16)] = jnp.zeros((16,), jnp.float32)
        return 0

    lax.fori_loop(0, E_CHUNK, zrow, 0)
    for k in range(ROWS_W // 128):
        pltpu.async_copy(ones_v, deg_sh.at[pl.ds(s * ROWS_W + k * 128, 128)],
                         dsem)
    pltpu.sync_copy(cidx_hbm.at[pl.ds(base, B_W)], cidx_v)
    pltpu.sync_copy(iidx_hbm.at[pl.ds(base, B_W)], iidx_v)
    for k in range(ROWS_W // 128):
        pltpu.make_async_copy(ones_v, deg_sh.at[pl.ds(s * ROWS_W, 128)],
                              dsem).wait()

    def orow(r, _):
        for j in range(D // 16):
            ones_v[r, pl.ds(j * 16, 16)] = jnp.ones((16,), jnp.float32)
        return 0

    lax.fori_loop(0, E_CHUNK, orow, 0)
    pltpu.sync_copy(dstr_hbm.at[pl.ds(w * E_STEPS, E_STEPS // 2)], didx)
    plsc.subcore_barrier()

    for h in range(2):
        if h == 1:
            pltpu.sync_copy(
                dstr_hbm.at[pl.ds(w * E_STEPS + E_STEPS // 2, E_STEPS // 2)],
                didx)
        for g in range(5):
            for j in range(8):
                pltpu.async_copy(ones_v, deg_sh.at[didx.at[g * 8 + j]],
                                 dsem, add=True)
            eb = (h * 5 + g) * _EC2
            pltpu.async_copy(ctab_hbm.at[cidx_v.at[pl.ds(eb, _EC2)]],
                             bufc, gsem).wait()
            pltpu.async_copy(itab_hbm.at[iidx_v.at[pl.ds(eb, _EC2)]],
                             bufi, gsem).wait()
            pltpu.sync_copy(mask_hbm.at[pl.ds(base + eb, _EC2)], mrow)

            def row(r, _):
                for j2 in range(D // 16):
                    sl = pl.ds(j2 * 16, 16)
                    m = mrow[r, sl]
                    bufc[r, sl] = bufi[r, sl] + m * (bufc[r, sl] - bufi[r, sl])
                return 0

            lax.fori_loop(0, _EC2, row, 0)
            pltpu.sync_copy(bufc, x_hbm.at[pl.ds(base + eb, _EC2)])
            for j in range(8):
                pltpu.make_async_copy(ones_v, deg_sh.at[didx.at[g * 8]],
                                      dsem).wait()

    plsc.subcore_barrier()

    off = s * ROWS_W
    pltpu.sync_copy(deg_sh.at[pl.ds(off, ROWS_W)],
                    deg_hbm.at[c, pl.ds(off, ROWS_W)])


_prep = pl.kernel(
    _prep_body,
    out_type=(jax.ShapeDtypeStruct((N_PAD, D), jnp.float32),
              jax.ShapeDtypeStruct((NC, N_ACC, D), jnp.float32)),
    mesh=_mesh,
    scratch_types=[
        pltpu.VMEM((B_W,), jnp.int32),
        pltpu.VMEM((B_W,), jnp.int32),
        pltpu.VMEM((_EC2, D), jnp.float32),
        pltpu.VMEM((_EC2, D), jnp.float32),
        pltpu.VMEM((_EC2, D), jnp.float32),
        pltpu.VMEM((40, E_CHUNK), jnp.int32),
        pltpu.VMEM((E_CHUNK, D), jnp.float32),
        pltpu.VMEM_SHARED((N_ACC, D), jnp.float32),
        pltpu.SemaphoreType.DMA,
        pltpu.SemaphoreType.DMA,
    ],
)


_HALF = E_STEPS // 2

_N0 = 112
_N1 = 48


def _edge_pipeline(xn_hbm, srcr_hbm, dstr_hbm, sidx, didx, rows, acc_sh,
                   gsem, ssem0, ssem1, base, n):
    ssems = (ssem0, ssem1)
    pending = [False, False]
    for off in range(0, n, _HALF):
        g = min(_HALF, n - off)
        for bb in (0, 1):
            if pending[bb]:
                pltpu.make_async_copy(rows.at[bb], acc_sh.at[didx.at[0]],
                                      ssems[bb]).wait()
                pending[bb] = False
        pltpu.sync_copy(srcr_hbm.at[pl.ds(base + off, g)],
                        sidx.at[pl.ds(0, g)])
        pltpu.sync_copy(dstr_hbm.at[pl.ds(base + off, g)],
                        didx.at[pl.ds(0, g)])
        pltpu.async_copy(xn_hbm.at[sidx.at[0]], rows.at[off % 2], gsem)
        for j in range(g):
            k = off + j
            b = k % 2
            pltpu.make_async_copy(xn_hbm.at[sidx.at[j]], rows.at[b],
                                  gsem).wait()
            pltpu.async_copy(rows.at[b], acc_sh.at[didx.at[j]], ssems[b],
                             add=True)
            pending[b] = True
            if j < g - 1:
                if pending[1 - b]:
                    pltpu.make_async_copy(rows.at[1 - b],
                                          acc_sh.at[didx.at[0]],
                                          ssems[1 - b]).wait()
                    pending[1 - b] = False
                pltpu.async_copy(xn_hbm.at[sidx.at[j + 1]], rows.at[1 - b],
                                 gsem)
    for bb in (0, 1):
        if pending[bb]:
            pltpu.make_async_copy(rows.at[bb], acc_sh.at[didx.at[0]],
                                  ssems[bb]).wait()


def _agg_body(xn_hbm, srcr_hbm, dstr_hbm, acc_hbm,
              sidx, didx, rows, acc_sh, gsem, ssem0, ssem1):
    c = lax.axis_index("c")
    s = lax.axis_index("s")

    def zero_row(r, _):
        for j in range(D // 16):
            rows[0, r, pl.ds(j * 16, 16)] = jnp.zeros((16,), jnp.float32)
        return 0

    lax.fori_loop(0, E_CHUNK, zero_row, 0)

    def zero_copy(k, _):
        off = s * ROWS_W + k * 128
        pltpu.sync_copy(rows.at[0], acc_sh.at[pl.ds(off, 128)])
        return 0

    lax.fori_loop(0, ROWS_W // 128, zero_copy, 0)

    plsc.subcore_barrier()

    @pl.when(c == 0)
    def _():
        _edge_pipeline(xn_hbm, srcr_hbm, dstr_hbm, sidx, didx, rows, acc_sh,
                       gsem, ssem0, ssem1, s * _N0, _N0)

    @pl.when(c == 1)
    def _():
        _edge_pipeline(xn_hbm, srcr_hbm, dstr_hbm, sidx, didx, rows, acc_sh,
                       gsem, ssem0, ssem1, NS * _N0 + s * _N1, _N1)

    plsc.subcore_barrier()

    off = s * ROWS_W
    pltpu.sync_copy(acc_sh.at[pl.ds(off, ROWS_W)],
                    acc_hbm.at[c, pl.ds(off, ROWS_W)])


_agg = pl.kernel(
    _agg_body,
    out_type=jax.ShapeDtypeStruct((NC, N_ACC, D), jnp.float32),
    mesh=_mesh,
    scratch_types=[
        pltpu.VMEM((_HALF, E_CHUNK), jnp.int32),
        pltpu.VMEM((_HALF, E_CHUNK), jnp.int32),
        pltpu.VMEM((2, E_CHUNK, D), jnp.float32),
        pltpu.VMEM_SHARED((N_ACC, D), jnp.float32),
        pltpu.SemaphoreType.DMA,
        pltpu.SemaphoreType.DMA,
        pltpu.SemaphoreType.DMA,
    ],
)


_RB = 1000


def _mm1_body(x_ref, ws_ref, wn_ref, b_ref, xs_ref, xn_ref):
    x = x_ref[...]
    xs_ref[...] = jnp.dot(x, ws_ref[...],
                          preferred_element_type=jnp.float32) + b_ref[...]
    xn_ref[...] = jnp.dot(x, wn_ref[...], preferred_element_type=jnp.float32)


def _mm2_body(xs1_ref, acc_ref, deg_ref, ws_ref, wn_ref, b_ref,
              xs2_ref, xn2_ref):
    d = deg_ref[0] + deg_ref[1]
    rdeg = 1.0 / jnp.maximum(d[:, 0:1], 1.0)
    agg = (acc_ref[0] + acc_ref[1]) * rdeg
    h = jnp.maximum(xs1_ref[...] + agg, 0.0)
    xs2_ref[...] = jnp.dot(h, ws_ref[...],
                           preferred_element_type=jnp.float32) + b_ref[...]
    xn2_ref[...] = jnp.dot(h, wn_ref[...], preferred_element_type=jnp.float32)


def _fin_body(xs2_ref, acc_ref, deg_ref, out_ref):
    d = deg_ref[0] + deg_ref[1]
    rdeg = 1.0 / jnp.maximum(d[:, 0:1], 1.0)
    out_ref[...] = xs2_ref[...] + (acc_ref[0] + acc_ref[1]) * rdeg


def _row_spec(block):
    return pl.BlockSpec(block, lambda i: (0,) * len(block)) if block[0] != _RB \
        else pl.BlockSpec(block, lambda i: (i,) + (0,) * (len(block) - 1))


def _mm1(x, ws, wn, b):
    return pl.pallas_call(
        _mm1_body,
        grid=(N_NODES // _RB,),
        in_specs=[
            pl.BlockSpec((_RB, D), lambda i: (i, 0)),
            pl.BlockSpec((D, D), lambda i: (0, 0)),
            pl.BlockSpec((D, D), lambda i: (0, 0)),
            pl.BlockSpec((1, D), lambda i: (0, 0)),
        ],
        out_specs=[pl.BlockSpec((_RB, D), lambda i: (i, 0)),
                   pl.BlockSpec((_RB, D), lambda i: (i, 0))],
        out_shape=[jax.ShapeDtypeStruct((N_NODES, D), jnp.float32),
                   jax.ShapeDtypeStruct((N_NODES, D), jnp.float32)],
    )(x, ws, wn, b)


def _mm2(xs1, acc, deg, ws, wn, b):
    return pl.pallas_call(
        _mm2_body,
        grid=(N_NODES // _RB,),
        in_specs=[
            pl.BlockSpec((_RB, D), lambda i: (i, 0)),
            pl.BlockSpec((NC, _RB, D), lambda i: (0, i, 0)),
            pl.BlockSpec((NC, _RB, D), lambda i: (0, i, 0)),
            pl.BlockSpec((D, D), lambda i: (0, 0)),
            pl.BlockSpec((D, D), lambda i: (0, 0)),
            pl.BlockSpec((1, D), lambda i: (0, 0)),
        ],
        out_specs=[pl.BlockSpec((_RB, D), lambda i: (i, 0)),
                   pl.BlockSpec((_RB, D), lambda i: (i, 0))],
        out_shape=[jax.ShapeDtypeStruct((N_NODES, D), jnp.float32),
                   jax.ShapeDtypeStruct((N_NODES, D), jnp.float32)],
    )(xs1, acc, deg, ws, wn, b)


def _fin(xs2, acc, deg):
    return pl.pallas_call(
        _fin_body,
        grid=(N_NODES // _RB,),
        in_specs=[
            pl.BlockSpec((_RB, D), lambda i: (i, 0)),
            pl.BlockSpec((NC, _RB, D), lambda i: (0, i, 0)),
            pl.BlockSpec((NC, _RB, D), lambda i: (0, i, 0)),
        ],
        out_specs=pl.BlockSpec((_RB, D), lambda i: (i, 0)),
        out_shape=jax.ShapeDtypeStruct((N_NODES, D), jnp.float32),
    )(xs2, acc, deg)


def kernel(node_ids, edge_index, client_table, item_table,
           W_self1, W_neigh1, b1, W_self2, W_neigh2, b2):
    ids = jnp.concatenate(
        [node_ids, jnp.zeros((N_PAD - N_NODES,), jnp.int32)])
    is_c = ids < NUM_CLIENTS
    cidx = jnp.where(is_c, ids, 0)
    iidx = jnp.where(is_c, 0, ids - NUM_CLIENTS)
    maskf = jnp.broadcast_to(is_c[:, None], (N_PAD, D)).astype(jnp.float32)

    pad_e = E_PAD - N_EDGES
    srcr = jnp.concatenate(
        [edge_index[0], jnp.zeros((pad_e,), jnp.int32)]
    ).reshape(NW * E_STEPS, E_CHUNK)
    dstr = jnp.concatenate(
        [edge_index[1], jnp.full((pad_e,), DUMMY_DST, jnp.int32)]
    ).reshape(NW * E_STEPS, E_CHUNK)

    b1r = b1.reshape(1, D)
    b2r = b2.reshape(1, D)

    x_pad, deg = _prep(cidx, iidx, maskf, client_table, item_table, dstr)
    x = x_pad[:N_NODES]
    xs1, xn1 = _mm1(x, W_self1, W_neigh1, b1r)
    acc1 = _agg(xn1, srcr, dstr)
    xs2, xn2 = _mm2(xs1, acc1, deg, W_self2, W_neigh2, b2r)
    acc2 = _agg(xn2, srcr, dstr)
    return _fin(xs2, acc2, deg)


# speedup vs baseline: 1.3846x; 1.0614x over previous
import functools

import jax
import jax.numpy as jnp
from jax import lax
from jax.experimental import pallas as pl
from jax.experimental.pallas import tpu as pltpu
from jax.experimental.pallas import tpu_sc as plsc

NUM_CLIENTS = 100000
N_NODES = 10000
N_EDGES = 320000
D = 128

NC, NS = 2, 16
NW = NC * NS
N_PAD = 10240
B_W = N_PAD // NW
E_W = N_EDGES // NW
E_CHUNK = 128
E_PAD = 327680
E_WP = E_PAD // NW
E_STEPS = E_WP // E_CHUNK
N_ACC = 10240
ROWS_W = N_ACC // NS
DUMMY_DST = N_ACC - 8

_mesh = plsc.VectorSubcoreMesh(core_axis_name="c", subcore_axis_name="s",
                               num_cores=NC, num_subcores=NS)


_EC2 = 32


def _prep_body(cidx_hbm, iidx_hbm, mask_hbm, ctab_hbm, itab_hbm, dstr_hbm,
               x_hbm, deg_hbm,
               cidx_v, iidx_v, bufc, bufi, mrow, didx, ones_v, deg_sh,
               gsem, dsem):
    c = lax.axis_index("c")
    s = lax.axis_index("s")
    w = s * NC + c
    base = w * B_W

    def zrow(r, _):
        for j in range(D // 16):
            ones_v[r, pl.ds(j * 16, 16)] = jnp.zeros((16,), jnp.float32)
        return 0

    lax.fori_loop(0, E_CHUNK, zrow, 0)
    for k in range(ROWS_W // 128):
        pltpu.async_copy(ones_v, deg_sh.at[pl.ds(s * ROWS_W + k * 128, 128)],
                         dsem)
    pltpu.sync_copy(cidx_hbm.at[pl.ds(base, B_W)], cidx_v)
    pltpu.sync_copy(iidx_hbm.at[pl.ds(base, B_W)], iidx_v)
    for k in range(ROWS_W // 128):
        pltpu.make_async_copy(ones_v, deg_sh.at[pl.ds(s * ROWS_W, 128)],
                              dsem).wait()

    def orow(r, _):
        for j in range(D // 16):
            ones_v[r, pl.ds(j * 16, 16)] = jnp.ones((16,), jnp.float32)
        return 0

    lax.fori_loop(0, E_CHUNK, orow, 0)
    pltpu.sync_copy(dstr_hbm.at[pl.ds(w * E_STEPS, E_STEPS // 2)], didx)
    plsc.subcore_barrier()

    for h in range(2):
        if h == 1:
            pltpu.sync_copy(
                dstr_hbm.at[pl.ds(w * E_STEPS + E_STEPS // 2, E_STEPS // 2)],
                didx)
        for g in range(5):
            for j in range(8):
                pltpu.async_copy(ones_v, deg_sh.at[didx.at[g * 8 + j]],
                                 dsem, add=True)
            eb = (h * 5 + g) * _EC2
            pltpu.async_copy(ctab_hbm.at[cidx_v.at[pl.ds(eb, _EC2)]],
                             bufc, gsem).wait()
            pltpu.async_copy(itab_hbm.at[iidx_v.at[pl.ds(eb, _EC2)]],
                             bufi, gsem).wait()
            pltpu.sync_copy(mask_hbm.at[pl.ds(base + eb, _EC2)], mrow)

            def row(r, _):
                for j2 in range(D // 16):
                    sl = pl.ds(j2 * 16, 16)
                    m = mrow[r, sl]
                    bufc[r, sl] = bufi[r, sl] + m * (bufc[r, sl] - bufi[r, sl])
                return 0

            lax.fori_loop(0, _EC2, row, 0)
            pltpu.sync_copy(bufc, x_hbm.at[pl.ds(base + eb, _EC2)])
            for j in range(8):
                pltpu.make_async_copy(ones_v, deg_sh.at[didx.at[g * 8]],
                                      dsem).wait()

    plsc.subcore_barrier()

    off = s * ROWS_W
    pltpu.sync_copy(deg_sh.at[pl.ds(off, ROWS_W)],
                    deg_hbm.at[c, pl.ds(off, ROWS_W)])


_prep = pl.kernel(
    _prep_body,
    out_type=(jax.ShapeDtypeStruct((N_PAD, D), jnp.float32),
              jax.ShapeDtypeStruct((NC, N_ACC, D), jnp.float32)),
    mesh=_mesh,
    scratch_types=[
        pltpu.VMEM((B_W,), jnp.int32),
        pltpu.VMEM((B_W,), jnp.int32),
        pltpu.VMEM((_EC2, D), jnp.float32),
        pltpu.VMEM((_EC2, D), jnp.float32),
        pltpu.VMEM((_EC2, D), jnp.float32),
        pltpu.VMEM((40, E_CHUNK), jnp.int32),
        pltpu.VMEM((E_CHUNK, D), jnp.float32),
        pltpu.VMEM_SHARED((N_ACC, D), jnp.float32),
        pltpu.SemaphoreType.DMA,
        pltpu.SemaphoreType.DMA,
    ],
)


_HALF = E_STEPS // 2

_N0 = 128
_N1 = 32


def _edge_pipeline(xn_hbm, srcr_hbm, dstr_hbm, sidx, didx, rows, acc_sh,
                   gsem, ssem0, ssem1, base, n):
    ssems = (ssem0, ssem1)
    pending = [False, False]
    for off in range(0, n, _HALF):
        g = min(_HALF, n - off)
        for bb in (0, 1):
            if pending[bb]:
                pltpu.make_async_copy(rows.at[bb], acc_sh.at[didx.at[0]],
                                      ssems[bb]).wait()
                pending[bb] = False
        pltpu.sync_copy(srcr_hbm.at[pl.ds(base + off, g)],
                        sidx.at[pl.ds(0, g)])
        pltpu.sync_copy(dstr_hbm.at[pl.ds(base + off, g)],
                        didx.at[pl.ds(0, g)])
        pltpu.async_copy(xn_hbm.at[sidx.at[0]], rows.at[off % 2], gsem)
        for j in range(g):
            k = off + j
            b = k % 2
            pltpu.make_async_copy(xn_hbm.at[sidx.at[j]], rows.at[b],
                                  gsem).wait()
            pltpu.async_copy(rows.at[b], acc_sh.at[didx.at[j]], ssems[b],
                             add=True)
            pending[b] = True
            if j < g - 1:
                if pending[1 - b]:
                    pltpu.make_async_copy(rows.at[1 - b],
                                          acc_sh.at[didx.at[0]],
                                          ssems[1 - b]).wait()
                    pending[1 - b] = False
                pltpu.async_copy(xn_hbm.at[sidx.at[j + 1]], rows.at[1 - b],
                                 gsem)
    for bb in (0, 1):
        if pending[bb]:
            pltpu.make_async_copy(rows.at[bb], acc_sh.at[didx.at[0]],
                                  ssems[bb]).wait()


def _agg_body(xn_hbm, srcr_hbm, dstr_hbm, acc_hbm,
              sidx, didx, rows, acc_sh, gsem, ssem0, ssem1):
    c = lax.axis_index("c")
    s = lax.axis_index("s")

    def zero_row(r, _):
        for j in range(D // 16):
            rows[0, r, pl.ds(j * 16, 16)] = jnp.zeros((16,), jnp.float32)
        return 0

    lax.fori_loop(0, E_CHUNK, zero_row, 0)

    def zero_copy(k, _):
        off = s * ROWS_W + k * 128
        pltpu.sync_copy(rows.at[0], acc_sh.at[pl.ds(off, 128)])
        return 0

    lax.fori_loop(0, ROWS_W // 128, zero_copy, 0)

    plsc.subcore_barrier()

    @pl.when(c == 0)
    def _():
        _edge_pipeline(xn_hbm, srcr_hbm, dstr_hbm, sidx, didx, rows, acc_sh,
                       gsem, ssem0, ssem1, s * _N0, _N0)

    @pl.when(c == 1)
    def _():
        _edge_pipeline(xn_hbm, srcr_hbm, dstr_hbm, sidx, didx, rows, acc_sh,
                       gsem, ssem0, ssem1, NS * _N0 + s * _N1, _N1)

    plsc.subcore_barrier()

    off = s * ROWS_W
    pltpu.sync_copy(acc_sh.at[pl.ds(off, ROWS_W)],
                    acc_hbm.at[c, pl.ds(off, ROWS_W)])


_agg = pl.kernel(
    _agg_body,
    out_type=jax.ShapeDtypeStruct((NC, N_ACC, D), jnp.float32),
    mesh=_mesh,
    scratch_types=[
        pltpu.VMEM((_HALF, E_CHUNK), jnp.int32),
        pltpu.VMEM((_HALF, E_CHUNK), jnp.int32),
        pltpu.VMEM((2, E_CHUNK, D), jnp.float32),
        pltpu.VMEM_SHARED((N_ACC, D), jnp.float32),
        pltpu.SemaphoreType.DMA,
        pltpu.SemaphoreType.DMA,
        pltpu.SemaphoreType.DMA,
    ],
)


_RB = 1000


def _mm1_body(x_ref, ws_ref, wn_ref, b_ref, xs_ref, xn_ref):
    x = x_ref[...]
    xs_ref[...] = jnp.dot(x, ws_ref[...],
                          preferred_element_type=jnp.float32) + b_ref[...]
    xn_ref[...] = jnp.dot(x, wn_ref[...], preferred_element_type=jnp.float32)


def _mm2_body(xs1_ref, acc_ref, deg_ref, ws_ref, wn_ref, b_ref,
              xs2_ref, xn2_ref):
    d = deg_ref[0] + deg_ref[1]
    rdeg = 1.0 / jnp.maximum(d[:, 0:1], 1.0)
    agg = (acc_ref[0] + acc_ref[1]) * rdeg
    h = jnp.maximum(xs1_ref[...] + agg, 0.0)
    xs2_ref[...] = jnp.dot(h, ws_ref[...],
                           preferred_element_type=jnp.float32) + b_ref[...]
    xn2_ref[...] = jnp.dot(h, wn_ref[...], preferred_element_type=jnp.float32)


def _fin_body(xs2_ref, acc_ref, deg_ref, out_ref):
    d = deg_ref[0] + deg_ref[1]
    rdeg = 1.0 / jnp.maximum(d[:, 0:1], 1.0)
    out_ref[...] = xs2_ref[...] + (acc_ref[0] + acc_ref[1]) * rdeg


def _row_spec(block):
    return pl.BlockSpec(block, lambda i: (0,) * len(block)) if block[0] != _RB \
        else pl.BlockSpec(block, lambda i: (i,) + (0,) * (len(block) - 1))


def _mm1(x, ws, wn, b):
    return pl.pallas_call(
        _mm1_body,
        grid=(N_NODES // _RB,),
        in_specs=[
            pl.BlockSpec((_RB, D), lambda i: (i, 0)),
            pl.BlockSpec((D, D), lambda i: (0, 0)),
            pl.BlockSpec((D, D), lambda i: (0, 0)),
            pl.BlockSpec((1, D), lambda i: (0, 0)),
        ],
        out_specs=[pl.BlockSpec((_RB, D), lambda i: (i, 0)),
                   pl.BlockSpec((_RB, D), lambda i: (i, 0))],
        out_shape=[jax.ShapeDtypeStruct((N_NODES, D), jnp.float32),
                   jax.ShapeDtypeStruct((N_NODES, D), jnp.float32)],
    )(x, ws, wn, b)


def _mm2(xs1, acc, deg, ws, wn, b):
    return pl.pallas_call(
        _mm2_body,
        grid=(N_NODES // _RB,),
        in_specs=[
            pl.BlockSpec((_RB, D), lambda i: (i, 0)),
            pl.BlockSpec((NC, _RB, D), lambda i: (0, i, 0)),
            pl.BlockSpec((NC, _RB, D), lambda i: (0, i, 0)),
            pl.BlockSpec((D, D), lambda i: (0, 0)),
            pl.BlockSpec((D, D), lambda i: (0, 0)),
            pl.BlockSpec((1, D), lambda i: (0, 0)),
        ],
        out_specs=[pl.BlockSpec((_RB, D), lambda i: (i, 0)),
                   pl.BlockSpec((_RB, D), lambda i: (i, 0))],
        out_shape=[jax.ShapeDtypeStruct((N_NODES, D), jnp.float32),
                   jax.ShapeDtypeStruct((N_NODES, D), jnp.float32)],
    )(xs1, acc, deg, ws, wn, b)


def _fin(xs2, acc, deg):
    return pl.pallas_call(
        _fin_body,
        grid=(N_NODES // _RB,),
        in_specs=[
            pl.BlockSpec((_RB, D), lambda i: (i, 0)),
            pl.BlockSpec((NC, _RB, D), lambda i: (0, i, 0)),
            pl.BlockSpec((NC, _RB, D), lambda i: (0, i, 0)),
        ],
        out_specs=pl.BlockSpec((_RB, D), lambda i: (i, 0)),
        out_shape=jax.ShapeDtypeStruct((N_NODES, D), jnp.float32),
    )(xs2, acc, deg)


def kernel(node_ids, edge_index, client_table, item_table,
           W_self1, W_neigh1, b1, W_self2, W_neigh2, b2):
    ids = jnp.concatenate(
        [node_ids, jnp.zeros((N_PAD - N_NODES,), jnp.int32)])
    is_c = ids < NUM_CLIENTS
    cidx = jnp.where(is_c, ids, 0)
    iidx = jnp.where(is_c, 0, ids - NUM_CLIENTS)
    maskf = jnp.broadcast_to(is_c[:, None], (N_PAD, D)).astype(jnp.float32)

    pad_e = E_PAD - N_EDGES
    srcr = jnp.concatenate(
        [edge_index[0], jnp.zeros((pad_e,), jnp.int32)]
    ).reshape(NW * E_STEPS, E_CHUNK)
    dstr = jnp.concatenate(
        [edge_index[1], jnp.full((pad_e,), DUMMY_DST, jnp.int32)]
    ).reshape(NW * E_STEPS, E_CHUNK)

    b1r = b1.reshape(1, D)
    b2r = b2.reshape(1, D)

    x_pad, deg = _prep(cidx, iidx, maskf, client_table, item_table, dstr)
    x = x_pad[:N_NODES]
    xs1, xn1 = _mm1(x, W_self1, W_neigh1, b1r)
    acc1 = _agg(xn1, srcr, dstr)
    xs2, xn2 = _mm2(xs1, acc1, deg, W_self2, W_neigh2, b2r)
    acc2 = _agg(xn2, srcr, dstr)
    return _fin(xs2, acc2, deg)
